# baseline (speedup 1.0000x reference)
import functools

import jax
import jax.numpy as jnp
from jax import lax
from jax.experimental import pallas as pl
from jax.experimental.pallas import tpu as pltpu

ALPHA = 0.5
EVENT_W = 1.0
CENS_W = 1.0
EPS = 1e-8


def _partials_kernel(preds_ref, onehot_ref, dur_row_ref, dur_col_ref,
                     ev_row_ref, out_ref, acc_ref, *, bi, bj, ni2, nj, num_t):
    c = pl.program_id(0)
    step = pl.program_id(1)
    i0 = (c * ni2 + step) * bi

    dur_i_row = dur_row_ref[:, pl.ds(i0, bi)]
    ev_i_row = ev_row_ref[:, pl.ds(i0, bi)]
    oh_blk = onehot_ref[:, pl.ds(i0, bi)]

    preds_i = preds_ref[pl.ds(i0, bi), :]
    d_blk = jnp.dot(preds_i, oh_blk,
                    preferred_element_type=jnp.float32)
    eye = (lax.broadcasted_iota(jnp.int32, (bi, bi), 0)
           == lax.broadcasted_iota(jnp.int32, (bi, bi), 1))
    g_row = jnp.sum(jnp.where(eye, d_blk, 0.0), axis=0,
                    keepdims=True)

    evf_row = jnp.where(ev_i_row == 1.0, 1.0, 0.0)
    w_row = jnp.where(ev_i_row == 1.0, EVENT_W, CENS_W)
    lik_p = jnp.sum(-jnp.log(g_row + EPS) * evf_row * w_row)

    g_eff = jnp.where(ev_i_row == 1.0, g_row, jnp.inf)

    racc = jnp.zeros((1, bi), jnp.float32)
    ccnt = jnp.zeros((1, bi), jnp.float32)
    for jq in range(nj):
        m_chunk = jnp.dot(preds_ref[jq * bj:(jq + 1) * bj, :], oh_blk,
                          preferred_element_type=jnp.float32)
        dur_j = dur_col_ref[jq * bj:(jq + 1) * bj, :]
        dgtf = jnp.where(dur_j > dur_i_row, 1.0, 0.0)
        contrib = jnp.maximum(m_chunk - g_eff, 0.0) * dgtf
        racc = racc + jnp.sum(contrib, axis=0, keepdims=True)
        ccnt = ccnt + jnp.sum(dgtf, axis=0, keepdims=True)

    @pl.when(step == 0)
    def _():
        acc_ref[0] = 0.0
        acc_ref[1] = 0.0
        acc_ref[2] = 0.0

    acc_ref[0] += lik_p
    acc_ref[1] += jnp.sum(racc)
    acc_ref[2] += jnp.sum(ccnt * evf_row)

    @pl.when(step == ni2 - 1)
    def _():
        lanes = lax.broadcasted_iota(jnp.int32, (1, 1, 128), 2)
        vec = jnp.where(lanes == 0, acc_ref[0],
                        jnp.where(lanes == 1, acc_ref[1],
                                  jnp.where(lanes == 2, acc_ref[2], 0.0)))
        out_ref[...] = vec


def _finish_kernel(p_ref, out_ref, *, n):
    lane = lax.broadcasted_iota(jnp.int32, (1, 128), 1)
    s = p_ref[0] + p_ref[1]
    lik_sum = jnp.sum(jnp.where(lane == 0, s, 0.0))
    rank_sum = jnp.sum(jnp.where(lane == 1, s, 0.0)) * EVENT_W
    cnt = jnp.sum(jnp.where(lane == 2, s, 0.0))
    rank = jnp.where(cnt > 0.0, rank_sum / jnp.maximum(cnt, 1.0), 0.0)
    res = ALPHA * (lik_sum / n) + (1.0 - ALPHA) * rank
    out_ref[...] = jnp.full((1, 128), res, dtype=jnp.float32)


def _deep_hit_loss(preds, targets, *, interpret=False):
    b, num_t = preds.shape
    bi = min(256, b // 2)
    ni2 = b // (2 * bi)
    bj = min(2048, b)
    nj = b // bj

    dur = targets[:, 0]
    ev = targets[:, 1]

    d_idx = (dur / jnp.max(dur) * (num_t - 1)).astype(jnp.int32)
    onehot = (jnp.arange(num_t, dtype=jnp.int32)[:, None]
              == d_idx[None, :]).astype(jnp.float32)

    dur_row = dur.reshape(1, b)
    dur_col = dur.reshape(b, 1)
    ev_row = ev.reshape(1, b)

    partials = pl.pallas_call(
        functools.partial(_partials_kernel, bi=bi, bj=bj, ni2=ni2, nj=nj,
                          num_t=num_t),
        grid=(2, ni2),
        in_specs=[
            pl.BlockSpec((b, num_t), lambda c, j: (0, 0)),
            pl.BlockSpec((num_t, b), lambda c, j: (0, 0)),
            pl.BlockSpec((1, b), lambda c, j: (0, 0)),
            pl.BlockSpec((b, 1), lambda c, j: (0, 0)),
            pl.BlockSpec((1, b), lambda c, j: (0, 0)),
        ],
        out_specs=pl.BlockSpec((1, 1, 128), lambda c, j: (c, 0, 0)),
        out_shape=jax.ShapeDtypeStruct((2, 1, 128), jnp.float32),
        scratch_shapes=[pltpu.SMEM((4,), jnp.float32)],
        compiler_params=pltpu.CompilerParams(
            dimension_semantics=("parallel", "arbitrary"),
            vmem_limit_bytes=44 * 1024 * 1024,
        ),
        name="deep_hit_partials",
        interpret=interpret,
    )(preds, onehot, dur_row, dur_col, ev_row)

    out = pl.pallas_call(
        functools.partial(_finish_kernel, n=float(b)),
        out_shape=jax.ShapeDtypeStruct((1, 128), jnp.float32),
        name="deep_hit_finish",
        interpret=interpret,
    )(partials)
    return out[0, 0]


def kernel(preds, targets):
    return _deep_hit_loss(preds, targets)


# speedup vs baseline: 1.0449x; 1.0449x over previous
import functools

import jax
import jax.numpy as jnp
from jax import lax
from jax.experimental import pallas as pl
from jax.experimental.pallas import tpu as pltpu

ALPHA = 0.5
EVENT_W = 1.0
CENS_W = 1.0
EPS = 1e-8


def _partials_kernel(preds_ref, onehot_ref, dur_row_ref, dur_col_ref,
                     ev_row_ref, out_ref, acc_ref, *, bi, bj, ni2, nj, num_t):
    c = pl.program_id(0)
    step = pl.program_id(1)
    i0 = (c * ni2 + step) * bi

    dur_i_row = dur_row_ref[:, pl.ds(i0, bi)]
    ev_i_row = ev_row_ref[:, pl.ds(i0, bi)]
    oh_blk = onehot_ref[:, pl.ds(i0, bi)]

    preds_i = preds_ref[pl.ds(i0, bi), :]
    d_blk = jnp.dot(preds_i, oh_blk,
                    preferred_element_type=jnp.float32)
    eye = (lax.broadcasted_iota(jnp.int32, (bi, bi), 0)
           == lax.broadcasted_iota(jnp.int32, (bi, bi), 1))
    g_row = jnp.sum(jnp.where(eye, d_blk, 0.0), axis=0,
                    keepdims=True)

    evf_row = jnp.where(ev_i_row == 1.0, 1.0, 0.0)
    w_row = jnp.where(ev_i_row == 1.0, EVENT_W, CENS_W)
    lik_p = jnp.sum(-jnp.log(g_row + EPS) * evf_row * w_row)

    g_eff = jnp.where(ev_i_row == 1.0, g_row, jnp.inf)

    racc = jnp.zeros((1, bi), jnp.float32)
    ccnt = jnp.zeros((1, bi), jnp.float32)
    for jq in range(nj):
        m_chunk = jnp.dot(preds_ref[jq * bj:(jq + 1) * bj, :], oh_blk,
                          preferred_element_type=jnp.float32)
        dur_j = dur_col_ref[jq * bj:(jq + 1) * bj, :]
        dgtf = jnp.where(dur_j > dur_i_row, 1.0, 0.0)
        contrib = jnp.maximum(m_chunk - g_eff, 0.0) * dgtf
        racc = racc + jnp.sum(contrib, axis=0, keepdims=True)
        ccnt = ccnt + jnp.sum(dgtf, axis=0, keepdims=True)

    @pl.when(step == 0)
    def _():
        acc_ref[0] = 0.0
        acc_ref[1] = 0.0
        acc_ref[2] = 0.0

    acc_ref[0] += lik_p
    acc_ref[1] += jnp.sum(racc)
    acc_ref[2] += jnp.sum(ccnt * evf_row)

    @pl.when(step == ni2 - 1)
    def _():
        lanes = lax.broadcasted_iota(jnp.int32, (1, 1, 128), 2)
        vec = jnp.where(lanes == 0, acc_ref[0],
                        jnp.where(lanes == 1, acc_ref[1],
                                  jnp.where(lanes == 2, acc_ref[2], 0.0)))
        out_ref[...] = vec


def _finish_kernel(p_ref, out_ref, *, n):
    lane = lax.broadcasted_iota(jnp.int32, (1, 128), 1)
    s = p_ref[0] + p_ref[1]
    lik_sum = jnp.sum(jnp.where(lane == 0, s, 0.0))
    rank_sum = jnp.sum(jnp.where(lane == 1, s, 0.0)) * EVENT_W
    cnt = jnp.sum(jnp.where(lane == 2, s, 0.0))
    rank = jnp.where(cnt > 0.0, rank_sum / jnp.maximum(cnt, 1.0), 0.0)
    res = ALPHA * (lik_sum / n) + (1.0 - ALPHA) * rank
    out_ref[...] = jnp.full((1, 128), res, dtype=jnp.float32)


def _deep_hit_loss(preds, targets, *, interpret=False):
    b, num_t = preds.shape
    bi = min(512, b // 2)
    ni2 = b // (2 * bi)
    bj = min(1024, b)
    nj = b // bj

    dur = targets[:, 0]
    ev = targets[:, 1]

    d_idx = (dur / jnp.max(dur) * (num_t - 1)).astype(jnp.int32)
    onehot = (jnp.arange(num_t, dtype=jnp.int32)[:, None]
              == d_idx[None, :]).astype(jnp.float32)

    dur_row = dur.reshape(1, b)
    dur_col = dur.reshape(b, 1)
    ev_row = ev.reshape(1, b)

    partials = pl.pallas_call(
        functools.partial(_partials_kernel, bi=bi, bj=bj, ni2=ni2, nj=nj,
                          num_t=num_t),
        grid=(2, ni2),
        in_specs=[
            pl.BlockSpec((b, num_t), lambda c, j: (0, 0)),
            pl.BlockSpec((num_t, b), lambda c, j: (0, 0)),
            pl.BlockSpec((1, b), lambda c, j: (0, 0)),
            pl.BlockSpec((b, 1), lambda c, j: (0, 0)),
            pl.BlockSpec((1, b), lambda c, j: (0, 0)),
        ],
        out_specs=pl.BlockSpec((1, 1, 128), lambda c, j: (c, 0, 0)),
        out_shape=jax.ShapeDtypeStruct((2, 1, 128), jnp.float32),
        scratch_shapes=[pltpu.SMEM((4,), jnp.float32)],
        compiler_params=pltpu.CompilerParams(
            dimension_semantics=("arbitrary", "arbitrary"),
            vmem_limit_bytes=44 * 1024 * 1024,
        ),
        name="deep_hit_partials",
        interpret=interpret,
    )(preds, onehot, dur_row, dur_col, ev_row)

    out = pl.pallas_call(
        functools.partial(_finish_kernel, n=float(b)),
        out_shape=jax.ShapeDtypeStruct((1, 128), jnp.float32),
        name="deep_hit_finish",
        interpret=interpret,
    )(partials)
    return out[0, 0]


def kernel(preds, targets):
    return _deep_hit_loss(preds, targets)


# speedup vs baseline: 1.0619x; 1.0163x over previous
import functools

import jax
import jax.numpy as jnp
from jax import lax
from jax.experimental import pallas as pl
from jax.experimental.pallas import tpu as pltpu

ALPHA = 0.5
EVENT_W = 1.0
CENS_W = 1.0
EPS = 1e-8


def _loss_kernel(preds_ref, onehot_ref, dur_row_ref, dur_col_ref,
                 ev_row_ref, out_ref, acc_ref, *, bi, bj, ni, nj, num_t):
    step = pl.program_id(0)
    i0 = step * bi

    dur_i_row = dur_row_ref[:, pl.ds(i0, bi)]
    ev_i_row = ev_row_ref[:, pl.ds(i0, bi)]
    oh_blk = onehot_ref[:, pl.ds(i0, bi)]

    preds_i = preds_ref[pl.ds(i0, bi), :]
    d_blk = jnp.dot(preds_i, oh_blk,
                    preferred_element_type=jnp.float32)
    eye = (lax.broadcasted_iota(jnp.int32, (bi, bi), 0)
           == lax.broadcasted_iota(jnp.int32, (bi, bi), 1))
    g_row = jnp.sum(jnp.where(eye, d_blk, 0.0), axis=0,
                    keepdims=True)

    evf_row = jnp.where(ev_i_row == 1.0, 1.0, 0.0)
    w_row = jnp.where(ev_i_row == 1.0, EVENT_W, CENS_W)
    lik_p = jnp.sum(-jnp.log(g_row + EPS) * evf_row * w_row)

    g_eff = jnp.where(ev_i_row == 1.0, g_row, jnp.inf)

    racc = jnp.zeros((1, bi), jnp.float32)
    ccnt = jnp.zeros((1, bi), jnp.float32)
    for jq in range(nj):
        m_chunk = jnp.dot(preds_ref[jq * bj:(jq + 1) * bj, :], oh_blk,
                          preferred_element_type=jnp.float32)
        dur_j = dur_col_ref[jq * bj:(jq + 1) * bj, :]
        dgtf = jnp.where(dur_j > dur_i_row, 1.0, 0.0)
        contrib = jnp.maximum(m_chunk - g_eff, 0.0) * dgtf
        racc = racc + jnp.sum(contrib, axis=0, keepdims=True)
        ccnt = ccnt + jnp.sum(dgtf, axis=0, keepdims=True)

    @pl.when(step == 0)
    def _():
        acc_ref[0] = 0.0
        acc_ref[1] = 0.0
        acc_ref[2] = 0.0

    acc_ref[0] += lik_p
    acc_ref[1] += jnp.sum(racc)
    acc_ref[2] += jnp.sum(ccnt * evf_row)

    @pl.when(step == ni - 1)
    def _():
        lik_sum = acc_ref[0]
        rank_sum = acc_ref[1] * EVENT_W
        cnt = acc_ref[2]
        rank = jnp.where(cnt > 0.0, rank_sum / jnp.maximum(cnt, 1.0), 0.0)
        res = ALPHA * (lik_sum / float(bi * ni)) + (1.0 - ALPHA) * rank
        out_ref[...] = jnp.full((1, 128), res, dtype=jnp.float32)


def _deep_hit_loss(preds, targets, *, interpret=False):
    b, num_t = preds.shape
    bi = min(512, b // 2)
    ni = b // bi
    bj = min(1024, b)
    nj = b // bj

    dur = targets[:, 0]
    ev = targets[:, 1]

    d_idx = (dur / jnp.max(dur) * (num_t - 1)).astype(jnp.int32)
    onehot = (jnp.arange(num_t, dtype=jnp.int32)[:, None]
              == d_idx[None, :]).astype(jnp.float32)

    dur_row = dur.reshape(1, b)
    dur_col = dur.reshape(b, 1)
    ev_row = ev.reshape(1, b)

    out = pl.pallas_call(
        functools.partial(_loss_kernel, bi=bi, bj=bj, ni=ni, nj=nj,
                          num_t=num_t),
        grid=(ni,),
        in_specs=[
            pl.BlockSpec((b, num_t), lambda i: (0, 0)),
            pl.BlockSpec((num_t, b), lambda i: (0, 0)),
            pl.BlockSpec((1, b), lambda i: (0, 0)),
            pl.BlockSpec((b, 1), lambda i: (0, 0)),
            pl.BlockSpec((1, b), lambda i: (0, 0)),
        ],
        out_specs=pl.BlockSpec((1, 128), lambda i: (0, 0)),
        out_shape=jax.ShapeDtypeStruct((1, 128), jnp.float32),
        scratch_shapes=[pltpu.SMEM((4,), jnp.float32)],
        compiler_params=pltpu.CompilerParams(
            dimension_semantics=("arbitrary",),
            vmem_limit_bytes=44 * 1024 * 1024,
        ),
        name="deep_hit_loss",
        interpret=interpret,
    )(preds, onehot, dur_row, dur_col, ev_row)
    return out[0, 0]


def kernel(preds, targets):
    return _deep_hit_loss(preds, targets)
